# baseline (speedup 1.0000x reference)
import functools

import jax
import jax.numpy as jnp
from jax import lax
from jax.experimental import pallas as pl
from jax.experimental.pallas import tpu as pltpu
from jax.experimental.pallas import tpu_sc as plsc

VOCAB = 1000000
DIM = 64
BATCH = 16384

NC = 2
NS = 16
LANES = 16
NW = NC * NS
B_PER_W = BATCH // NW
CHUNK = 128
N_CHUNKS = B_PER_W // CHUNK
GROUPS = B_PER_W // LANES

_SC_PARAMS = pltpu.CompilerParams(
    needs_layout_passes=False, use_tc_tiling_on_sc=False)
_SC_MESH = plsc.VectorSubcoreMesh(core_axis_name="c", subcore_axis_name="s")


def _worker_base():
    wid = lax.axis_index("s") * NC + lax.axis_index("c")
    return wid * B_PER_W


def _gather_rows(idx_v, table_hbm, rows_v, sem):
    copies = []
    for j in range(N_CHUNKS):
        sl = pl.ds(j * CHUNK, CHUNK)
        copies.append(
            pltpu.async_copy(table_hbm.at[idx_v.at[sl]], rows_v.at[sl], sem))
    return copies


def _sc_gather_kernel(idx_hbm, table_hbm, out_hbm, idx_v, rows_v, sem):
    base = _worker_base()
    pltpu.sync_copy(idx_hbm.at[pl.ds(base, B_PER_W)], idx_v)
    for cp in _gather_rows(idx_v, table_hbm, rows_v, sem):
        cp.wait()
    pltpu.sync_copy(rows_v, out_hbm.at[pl.ds(base, B_PER_W)])


_sc_gather = functools.partial(
    pl.kernel,
    mesh=_SC_MESH,
    out_type=jax.ShapeDtypeStruct((BATCH, DIM), jnp.float32),
    scratch_types=[
        pltpu.VMEM((B_PER_W,), jnp.int32),
        pltpu.VMEM((B_PER_W, DIM), jnp.float32),
        pltpu.SemaphoreType.DMA,
    ],
    compiler_params=_SC_PARAMS,
)(_sc_gather_kernel)


def _sc_dot_kernel(idx_hbm, table_hbm, rows_a_hbm, out_hbm,
                   idx_v, a_v, b_v, tr_v, logit_v, sem_a, sem_b):
    base = _worker_base()
    pltpu.sync_copy(idx_hbm.at[pl.ds(base, B_PER_W)], idx_v)
    copies = _gather_rows(idx_v, table_hbm, b_v, sem_b)
    pltpu.sync_copy(rows_a_hbm.at[pl.ds(base, B_PER_W)], a_v)
    for cp in copies:
        cp.wait()

    lane = lax.iota(jnp.int32, LANES)
    lane16 = lane * LANES

    def group_body(g, _):
        row0 = g * LANES
        for r in range(LANES):
            row = row0 + r
            s = (a_v[row, pl.ds(0, LANES)] * b_v[row, pl.ds(0, LANES)]
                 + a_v[row, pl.ds(LANES, LANES)] * b_v[row, pl.ds(LANES, LANES)]
                 + a_v[row, pl.ds(2 * LANES, LANES)] * b_v[row, pl.ds(2 * LANES, LANES)]
                 + a_v[row, pl.ds(3 * LANES, LANES)] * b_v[row, pl.ds(3 * LANES, LANES)])
            plsc.store_scatter(tr_v, [lane16 + r], s)
        acc = tr_v[pl.ds(0, LANES)]
        for j in range(1, LANES):
            acc = acc + tr_v[pl.ds(j * LANES, LANES)]
        logit_v[pl.ds(row0, LANES)] = acc
        return 0

    lax.fori_loop(0, GROUPS, group_body, 0)

    pltpu.sync_copy(logit_v, out_hbm.at[pl.ds(base, B_PER_W)])


_sc_dot = functools.partial(
    pl.kernel,
    mesh=_SC_MESH,
    out_type=jax.ShapeDtypeStruct((BATCH,), jnp.float32),
    scratch_types=[
        pltpu.VMEM((B_PER_W,), jnp.int32),
        pltpu.VMEM((B_PER_W, DIM), jnp.float32),
        pltpu.VMEM((B_PER_W, DIM), jnp.float32),
        pltpu.VMEM((LANES * LANES,), jnp.float32),
        pltpu.VMEM((B_PER_W,), jnp.float32),
        pltpu.SemaphoreType.DMA,
        pltpu.SemaphoreType.DMA,
    ],
    compiler_params=_SC_PARAMS,
)(_sc_dot_kernel)


def _bce_kernel(logits_ref, labels_ref, out_ref):
    x = logits_ref[...]
    y = labels_ref[...]
    per = jnp.maximum(x, 0.0) - x * y + jnp.log1p(jnp.exp(-jnp.abs(x)))
    out_ref[0, 0] = jnp.sum(per) / BATCH


def kernel(center_words, context_words, labels, W_in, W_out):
    rows_a = _sc_gather(center_words.astype(jnp.int32), W_in)
    logits = _sc_dot(context_words.astype(jnp.int32), W_out, rows_a)
    loss = pl.pallas_call(
        _bce_kernel,
        out_shape=jax.ShapeDtypeStruct((1, 1), jnp.float32),
        in_specs=[
            pl.BlockSpec(memory_space=pltpu.VMEM),
            pl.BlockSpec(memory_space=pltpu.VMEM),
        ],
        out_specs=pl.BlockSpec(memory_space=pltpu.SMEM),
    )(logits.reshape(128, 128), labels.reshape(128, 128))
    return loss[0, 0]
